# baseline (speedup 1.0000x reference)
import jax
import jax.numpy as jnp
from jax.experimental import pallas as pl

_GRID = 25


def _copy_body(xb, eb, xob, eob):
    xob[...] = xb[...]
    eob[...] = eb[...]


def kernel(x, edge_index, edge_attr):
    del edge_index
    e64 = edge_attr.reshape(edge_attr.shape[0] // 4, 64)
    xb = x.shape[0] // _GRID
    eb = e64.shape[0] // _GRID
    xo, eo = pl.pallas_call(
        _copy_body,
        grid=(_GRID,),
        in_specs=[
            pl.BlockSpec((xb, x.shape[1]), lambda i: (i, 0)),
            pl.BlockSpec((eb, 64), lambda i: (i, 0)),
        ],
        out_specs=[
            pl.BlockSpec((xb, x.shape[1]), lambda i: (i, 0)),
            pl.BlockSpec((eb, 64), lambda i: (i, 0)),
        ],
        out_shape=[
            jax.ShapeDtypeStruct(x.shape, x.dtype),
            jax.ShapeDtypeStruct(e64.shape, e64.dtype),
        ],
    )(x, e64)
    return (xo, eo.reshape(edge_attr.shape))


# speedup vs baseline: 1.2225x; 1.2225x over previous
import jax
import jax.numpy as jnp
from jax.experimental import pallas as pl
from jax.experimental.pallas import tpu as pltpu

_K = 8
_CH = 4000
_CPC = 10


def _copy_body(x_ref, e_ref, xo_ref, eo_ref, xbuf, ebuf, xsi, xso, esi, eso):
    per_chain = _CH * _CPC

    xin = pltpu.make_async_copy(x_ref, xbuf, xsi)
    xin.start()

    def ein(k, c):
        return pltpu.make_async_copy(
            e_ref.at[pl.ds(k * per_chain + c * _CH, _CH)],
            ebuf.at[k, c % 2],
            esi.at[k, c % 2],
        )

    def eout(k, c):
        return pltpu.make_async_copy(
            ebuf.at[k, c % 2],
            eo_ref.at[pl.ds(k * per_chain + c * _CH, _CH)],
            eso.at[k, c % 2],
        )

    for k in range(_K):
        ein(k, 0).start()
    xin.wait()
    xout = pltpu.make_async_copy(xbuf, xo_ref, xso)
    xout.start()
    for c in range(_CPC):
        for k in range(_K):
            if c + 1 < _CPC:
                if c >= 1:
                    eout(k, c - 1).wait()
                ein(k, c + 1).start()
            ein(k, c).wait()
            eout(k, c).start()
    for k in range(_K):
        if _CPC >= 2:
            eout(k, _CPC - 2).wait()
        eout(k, _CPC - 1).wait()
    xout.wait()


def kernel(x, edge_index, edge_attr):
    del edge_index
    xo, eo = pl.pallas_call(
        _copy_body,
        in_specs=[
            pl.BlockSpec(memory_space=pl.ANY),
            pl.BlockSpec(memory_space=pl.ANY),
        ],
        out_specs=[
            pl.BlockSpec(memory_space=pl.ANY),
            pl.BlockSpec(memory_space=pl.ANY),
        ],
        out_shape=[
            jax.ShapeDtypeStruct(x.shape, x.dtype),
            jax.ShapeDtypeStruct(edge_attr.shape, edge_attr.dtype),
        ],
        scratch_shapes=[
            pltpu.VMEM(x.shape, x.dtype),
            pltpu.VMEM((_K, 2, _CH, edge_attr.shape[1]), edge_attr.dtype),
            pltpu.SemaphoreType.DMA,
            pltpu.SemaphoreType.DMA,
            pltpu.SemaphoreType.DMA((_K, 2)),
            pltpu.SemaphoreType.DMA((_K, 2)),
        ],
    )(x, edge_attr)
    return (xo, eo)


# speedup vs baseline: 1.8891x; 1.5453x over previous
import jax
import jax.numpy as jnp
from jax.experimental import pallas as pl
from jax.experimental.pallas import tpu as pltpu

_GRID = 10


def _copy_body(x_ref, e_ref, xo_ref, eo_ref):
    eo_ref[...] = e_ref[...]
    xo_ref[...] = x_ref[...]


def kernel(x, edge_index, edge_attr):
    del edge_index
    xb = x.shape[0] // _GRID
    xo, eo = pl.pallas_call(
        _copy_body,
        grid=(_GRID,),
        in_specs=[
            pl.BlockSpec((xb, x.shape[1]), lambda i: (i, 0)),
            pl.BlockSpec((8, edge_attr.shape[1]), lambda i: (0, 0)),
        ],
        out_specs=[
            pl.BlockSpec((xb, x.shape[1]), lambda i: (i, 0)),
            pl.BlockSpec((8, edge_attr.shape[1]), lambda i: (0, 0)),
        ],
        out_shape=[
            jax.ShapeDtypeStruct(x.shape, x.dtype),
            jax.ShapeDtypeStruct(edge_attr.shape, edge_attr.dtype),
        ],
        input_output_aliases={1: 1},
    )(x, edge_attr)
    return (xo, eo)


# speedup vs baseline: 1.9233x; 1.0181x over previous
import jax
import jax.numpy as jnp
from jax.experimental import pallas as pl


def _copy_body(x_ref, e_ref, xo_ref, eo_ref):
    xo_ref[...] = x_ref[...]
    eo_ref[...] = e_ref[...]


def kernel(x, edge_index, edge_attr):
    del edge_index
    xo, eo = pl.pallas_call(
        _copy_body,
        grid=(1,),
        in_specs=[
            pl.BlockSpec((8, x.shape[1]), lambda i: (0, 0)),
            pl.BlockSpec((8, edge_attr.shape[1]), lambda i: (0, 0)),
        ],
        out_specs=[
            pl.BlockSpec((8, x.shape[1]), lambda i: (0, 0)),
            pl.BlockSpec((8, edge_attr.shape[1]), lambda i: (0, 0)),
        ],
        out_shape=[
            jax.ShapeDtypeStruct(x.shape, x.dtype),
            jax.ShapeDtypeStruct(edge_attr.shape, edge_attr.dtype),
        ],
        input_output_aliases={0: 0, 1: 1},
    )(x, edge_attr)
    return (xo, eo)
